# Initial kernel scaffold; baseline (speedup 1.0000x reference)
import jax
import jax.numpy as jnp
from jax.experimental import pallas as pl


def kernel(x, edge_index, W_expand, b_expand, W_gcn, b_gcn, bn_gamma, bn_beta, static_res, static_post, W_dyn_res, w_dyn_post, W_out, b_out):
    raise NotImplementedError("write your pallas kernel here")


# speedup vs baseline: 4.1212x; 4.1212x over previous
import functools

import jax
import jax.numpy as jnp
from jax import lax
from jax.experimental import pallas as pl
from jax.experimental.pallas import tpu as pltpu
from jax.experimental.pallas import tpu_sc as plsc

N = 10000
E = 320000
HID = 128
NS = 4
TAU = 0.1
SINK_ITERS = 10

NC = 2
NT = 16
NW = NC * NT

C = 128
K = 80
EP = NW * K * C
ACC = 10240
RPT = ACC // NT

B = 1000
NB = N // B


def _mesh():
    return plsc.VectorSubcoreMesh(
        core_axis_name="c", subcore_axis_name="s", num_cores=NC, num_subcores=NT
    )


def _deg_body(dst_hbm, out_hbm, idx_v, ones_v, buf_v, acc_s):
    cid = lax.axis_index("c")
    sid = lax.axis_index("s")
    wid = sid * NC + cid
    for j in range(C // 16):
        ones_v[pl.ds(j * 16, 16)] = jnp.ones((16,), jnp.float32)
    for j in range(RPT // 16):
        buf_v[pl.ds(j * 16, 16)] = jnp.zeros((16,), jnp.float32)
    pltpu.sync_copy(buf_v, acc_s.at[pl.ds(sid * RPT, RPT)])
    pltpu.sync_copy(dst_hbm.at[pl.ds(wid * K, K)], idx_v)
    plsc.subcore_barrier()

    def body(j, carry):
        pltpu.sync_copy(ones_v, acc_s.at[idx_v.at[j]], add=True)
        return carry

    lax.fori_loop(0, K, body, 0)
    plsc.subcore_barrier()
    pltpu.sync_copy(acc_s.at[pl.ds(sid * RPT, RPT)], buf_v)
    pltpu.sync_copy(buf_v, out_hbm.at[cid, pl.ds(sid * RPT, RPT)])


def _deg_call(dst2d):
    fn = functools.partial(
        pl.kernel,
        out_type=jax.ShapeDtypeStruct((NC, ACC), jnp.float32),
        mesh=_mesh(),
        scratch_types=[
            pltpu.VMEM((K, C), jnp.int32),
            pltpu.VMEM((C,), jnp.float32),
            pltpu.VMEM((RPT,), jnp.float32),
            pltpu.VMEM_SHARED((ACC,), jnp.float32),
        ],
    )(_deg_body)
    return fn(dst2d)


HALFN = N // NC
ACC_R = 5120
RPT_R = ACC_R // NT
KT = NW * K // NT


def _msg_body(g_hbm, src_hbm, dst_hbm, out_hbm, si_v, di_v, b0, b1, z_v,
              acc_s, sem0, sem1):
    cid = lax.axis_index("c")
    sid = lax.axis_index("s")
    base = cid * HALFN
    for i in range(16):
        for j in range(HID // 16):
            z_v[i, pl.ds(j * 16, 16)] = jnp.zeros((16,), jnp.float32)

    def zbody(k, carry):
        pltpu.sync_copy(z_v, acc_s.at[pl.ds(sid * RPT_R + k * 16, 16)])
        return carry

    lax.fori_loop(0, RPT_R // 16, zbody, 0)
    pltpu.sync_copy(src_hbm.at[pl.ds(sid * KT, KT)], si_v)
    pltpu.sync_copy(dst_hbm.at[pl.ds(sid * KT, KT)], di_v)

    iota = lax.iota(jnp.int32, 16)

    def rwj(j, carry):
        def rwm(m, c2):
            d = di_v[j, pl.ds(m * 16, 16)]
            t = d - base
            ok = (t >= 0) & (t < HALFN)
            dum = HALFN + ((iota * 7 + m) % (ACC_R - HALFN))
            di_v[j, pl.ds(m * 16, 16)] = jnp.where(ok, t, dum)
            return c2

        return lax.fori_loop(0, C // 16, rwm, carry)

    lax.fori_loop(0, KT, rwj, 0)
    plsc.subcore_barrier()

    def mbody(j, carry):
        pltpu.async_copy(g_hbm.at[si_v.at[2 * j]], b0, sem0).wait()
        pltpu.sync_copy(b0, acc_s.at[di_v.at[2 * j]], add=True)
        pltpu.async_copy(g_hbm.at[si_v.at[2 * j + 1]], b1, sem1).wait()
        pltpu.sync_copy(b1, acc_s.at[di_v.at[2 * j + 1]], add=True)
        return carry

    lax.fori_loop(0, KT // 2, mbody, 0)
    plsc.subcore_barrier()

    off = 0
    for sz in (C, C, RPT_R - 2 * C):
        pltpu.sync_copy(acc_s.at[pl.ds(sid * RPT_R + off, sz)],
                        b0.at[pl.ds(0, sz)])
        pltpu.sync_copy(b0.at[pl.ds(0, sz)],
                        out_hbm.at[cid, pl.ds(sid * RPT_R + off, sz)])
        off += sz


def _msg_call(g, src2d, dst2d):
    fn = functools.partial(
        pl.kernel,
        out_type=jax.ShapeDtypeStruct((NC, ACC_R, HID), jnp.float32),
        mesh=_mesh(),
        scratch_types=[
            pltpu.VMEM((KT, C), jnp.int32),
            pltpu.VMEM((KT, C), jnp.int32),
            pltpu.VMEM((C, HID), jnp.float32),
            pltpu.VMEM((C, HID), jnp.float32),
            pltpu.VMEM((16, HID), jnp.float32),
            pltpu.VMEM_SHARED((ACC_R, HID), jnp.float32),
            pltpu.SemaphoreType.DMA,
            pltpu.SemaphoreType.DMA,
        ],
    )(_msg_body)
    return fn(g, src2d, dst2d)


def _expand_kernel(x_ref, we_ref, be_ref, wg0_ref, degt_ref,
                   xs_ref, sum_ref, h_ref, g_ref):
    i = pl.program_id(0)
    xs = jnp.dot(x_ref[...], we_ref[...], preferred_element_type=jnp.float32)
    xs = xs + be_ref[...]
    xs_ref[...] = xs

    @pl.when(i == 0)
    def _():
        sum_ref[...] = jnp.zeros_like(sum_ref)

    sum_ref[...] += jnp.sum(xs, axis=0, keepdims=True)
    v = xs.reshape(B, NS, HID)
    x_agg = jnp.mean(v, axis=1)
    h = jnp.dot(x_agg, wg0_ref[...], preferred_element_type=jnp.float32)
    h_ref[...] = h
    deg = degt_ref[:, 0] + degt_ref[:, 1] + 1.0
    dinv = lax.rsqrt(jnp.maximum(deg, 1.0))
    g_ref[...] = h * dinv[:, None]


def _expand_call(x, W_expand, b2, Wg0, degt):
    return pl.pallas_call(
        _expand_kernel,
        grid=(NB,),
        in_specs=[
            pl.BlockSpec((B, HID), lambda i: (i, 0)),
            pl.BlockSpec((HID, NS * HID), lambda i: (0, 0)),
            pl.BlockSpec((1, NS * HID), lambda i: (0, 0)),
            pl.BlockSpec((HID, HID), lambda i: (0, 0)),
            pl.BlockSpec((B, NC), lambda i: (i, 0)),
        ],
        out_specs=[
            pl.BlockSpec((B, NS * HID), lambda i: (i, 0)),
            pl.BlockSpec((1, NS * HID), lambda i: (0, 0)),
            pl.BlockSpec((B, HID), lambda i: (i, 0)),
            pl.BlockSpec((B, HID), lambda i: (i, 0)),
        ],
        out_shape=[
            jax.ShapeDtypeStruct((N, NS * HID), jnp.float32),
            jax.ShapeDtypeStruct((1, NS * HID), jnp.float32),
            jax.ShapeDtypeStruct((N, HID), jnp.float32),
            jax.ShapeDtypeStruct((N, HID), jnp.float32),
        ],
    )(x, W_expand, b2, Wg0, degt)


def _mix_kernel(last, agg_ref, h_ref, degt_ref, xs_ref, sumxs_ref, bg_ref,
                gam_ref, bet_ref, sres_ref, spost_ref, wdr_ref, wdp_ref,
                wn_ref, bn2_ref, *rest):
    if last:
        (out_ref, r_scr, st_scr, hres_scr, hpost_scr, mu_scr) = rest
    else:
        (xsn_ref, sumn_ref, hn_ref, gn_ref,
         r_scr, st_scr, hres_scr, hpost_scr, mu_scr) = rest
    p = pl.program_id(0)
    i = pl.program_id(1)

    @pl.when(p == 0)
    def _phase0():
        deg = degt_ref[:, 0] + degt_ref[:, 1] + 1.0
        dinv = lax.rsqrt(jnp.maximum(deg, 1.0))
        agg = agg_ref[0]
        r = dinv[:, None] * agg + h_ref[...] * (dinv * dinv)[:, None] + bg_ref[...]
        r = jnp.maximum(r, 0.0)
        r_scr[pl.ds(i * B, B), :] = r

        @pl.when(i == 0)
        def _():
            st_scr[...] = jnp.zeros_like(st_scr)

        st_scr[0:1, :] += jnp.sum(r, axis=0, keepdims=True)
        st_scr[1:2, :] += jnp.sum(r * r, axis=0, keepdims=True)

        @pl.when(i == NB - 1)
        def _fin():
            mean = st_scr[0:1, :] * (1.0 / N)
            ex2 = st_scr[1:2, :] * (1.0 / N)
            var = ex2 - mean * mean
            mu_scr[0:1, :] = mean
            mu_scr[1:2, :] = lax.rsqrt(var + 1e-5)
            nxm = sumxs_ref[...].reshape(NS, HID) * (1.0 / N)
            nrm = jnp.sqrt(jnp.sum(nxm * nxm, axis=1, keepdims=True)) + 1e-6
            nx = nxm / nrm
            dyn_res = jnp.tanh(
                jnp.dot(nx, wdr_ref[...], preferred_element_type=jnp.float32))
            dyn_post = jnp.tanh(
                jnp.dot(nx, wdp_ref[...], preferred_element_type=jnp.float32))
            M = jnp.exp((sres_ref[...] + dyn_res) * (1.0 / TAU))
            for _ in range(SINK_ITERS):
                M = M / (jnp.sum(M, axis=1, keepdims=True) + 1e-8)
                M = M / (jnp.sum(M, axis=0, keepdims=True) + 1e-8)
            hres_scr[...] = M
            z = spost_ref[...] + dyn_post
            hpost_scr[...] = 1.0 / (1.0 + jnp.exp(-z))

    @pl.when(p == 1)
    def _phase1():
        r = r_scr[pl.ds(i * B, B), :]
        m = gam_ref[...] * (r - mu_scr[0:1, :]) * mu_scr[1:2, :] + bet_ref[...]
        v = xs_ref[...].reshape(B, NS, HID)
        H = hres_scr[...]
        hp = hpost_scr[...]
        outs = []
        for a in range(NS):
            acc = hp[a, 0] * m
            for b in range(NS):
                acc = acc + H[a, b] * v[:, b, :]
            outs.append(acc)
        x_agg = (outs[0] + outs[1] + outs[2] + outs[3]) * (1.0 / NS)
        if last:
            out_ref[...] = jnp.dot(
                x_agg, wn_ref[...], preferred_element_type=jnp.float32
            ) + bn2_ref[...]
        else:
            xs_new = jnp.stack(outs, axis=1)
            xsn_ref[...] = xs_new.reshape(B, NS * HID)

            @pl.when(i == 0)
            def _():
                sumn_ref[...] = jnp.zeros_like(sumn_ref)

            sumn_ref[...] += jnp.sum(xs_new, axis=0).reshape(1, NS * HID)
            hn = jnp.dot(x_agg, wn_ref[...], preferred_element_type=jnp.float32)
            hn_ref[...] = hn
            deg = degt_ref[:, 0] + degt_ref[:, 1] + 1.0
            dinv = lax.rsqrt(jnp.maximum(deg, 1.0))
            gn_ref[...] = hn * dinv[:, None]


def _mix_call(last, agg, h, degt, xs, sumxs, bg, gam, bet, sres, spost, wdr,
              wdp, wn, bn2):
    wn_cols = wn.shape[1]
    in_specs = [
        pl.BlockSpec((1, B, HID),
                     lambda p, i: ((1 - p) * (i // (NB // NC)),
                                   (1 - p) * (i % (NB // NC)), 0)),
        pl.BlockSpec((B, HID), lambda p, i: (i * (1 - p), 0)),
        pl.BlockSpec((B, NC), lambda p, i: (i, 0)),
        pl.BlockSpec((B, NS * HID), lambda p, i: (i * p, 0)),
        pl.BlockSpec((1, NS * HID), lambda p, i: (0, 0)),
        pl.BlockSpec((1, HID), lambda p, i: (0, 0)),
        pl.BlockSpec((1, HID), lambda p, i: (0, 0)),
        pl.BlockSpec((1, HID), lambda p, i: (0, 0)),
        pl.BlockSpec((NS, NS), lambda p, i: (0, 0)),
        pl.BlockSpec((NS, 1), lambda p, i: (0, 0)),
        pl.BlockSpec((HID, NS), lambda p, i: (0, 0)),
        pl.BlockSpec((HID, 1), lambda p, i: (0, 0)),
        pl.BlockSpec((HID, wn_cols), lambda p, i: (0, 0)),
        pl.BlockSpec((1, wn_cols), lambda p, i: (0, 0)),
    ]
    if last:
        out_specs = [pl.BlockSpec((B, wn_cols), lambda p, i: (i * p, 0))]
        out_shape = [jax.ShapeDtypeStruct((N, wn_cols), jnp.float32)]
    else:
        out_specs = [
            pl.BlockSpec((B, NS * HID), lambda p, i: (i * p, 0)),
            pl.BlockSpec((1, NS * HID), lambda p, i: (0, 0)),
            pl.BlockSpec((B, HID), lambda p, i: (i * p, 0)),
            pl.BlockSpec((B, HID), lambda p, i: (i * p, 0)),
        ]
        out_shape = [
            jax.ShapeDtypeStruct((N, NS * HID), jnp.float32),
            jax.ShapeDtypeStruct((1, NS * HID), jnp.float32),
            jax.ShapeDtypeStruct((N, HID), jnp.float32),
            jax.ShapeDtypeStruct((N, HID), jnp.float32),
        ]
    return pl.pallas_call(
        functools.partial(_mix_kernel, last),
        grid=(2, NB),
        in_specs=in_specs,
        out_specs=out_specs,
        out_shape=out_shape,
        scratch_shapes=[
            pltpu.VMEM((N, HID), jnp.float32),
            pltpu.VMEM((2, HID), jnp.float32),
            pltpu.VMEM((NS, NS), jnp.float32),
            pltpu.VMEM((NS, 1), jnp.float32),
            pltpu.VMEM((2, HID), jnp.float32),
        ],
    )(agg, h, degt, xs, sumxs, bg, gam, bet, sres, spost, wdr, wdp, wn, bn2)


def kernel(x, edge_index, W_expand, b_expand, W_gcn, b_gcn, bn_gamma, bn_beta,
           static_res, static_post, W_dyn_res, w_dyn_post, W_out, b_out):
    src = edge_index[0]
    dst = edge_index[1]
    pad = EP - E
    src2d = jnp.concatenate([src, jnp.zeros((pad,), jnp.int32)]).reshape(NW * K, C)
    dst2d = jnp.concatenate([dst, jnp.full((pad,), N, jnp.int32)]).reshape(NW * K, C)

    degp = _deg_call(dst2d)
    degt = jnp.transpose(degp)
    xs, sumxs, h, g = _expand_call(
        x, W_expand, b_expand.reshape(1, NS * HID), W_gcn[0], degt)

    for l in range(3):
        agg = _msg_call(g, src2d, dst2d)
        last = l == 2
        wn = W_out if last else W_gcn[l + 1]
        bn2 = b_out.reshape(1, -1) if last else jnp.zeros((1, HID), jnp.float32)
        res = _mix_call(
            last, agg, h, degt, xs, sumxs,
            b_gcn[l].reshape(1, HID),
            bn_gamma[l].reshape(1, HID),
            bn_beta[l].reshape(1, HID),
            static_res[l],
            static_post[l].reshape(NS, 1),
            W_dyn_res[l],
            w_dyn_post[l].reshape(HID, 1),
            wn, bn2)
        if last:
            return res[0]
        xs, sumxs, h, g = res


# speedup vs baseline: 4.5156x; 1.0957x over previous
import functools

import jax
import jax.numpy as jnp
from jax import lax
from jax.experimental import pallas as pl
from jax.experimental.pallas import tpu as pltpu
from jax.experimental.pallas import tpu_sc as plsc

N = 10000
E = 320000
HID = 128
NS = 4
TAU = 0.1
SINK_ITERS = 10

NC = 2
NT = 16
NW = NC * NT

C = 128
K = 80
EP = NW * K * C
ACC = 10240
RPT = ACC // NT

B = 1000
NB = N // B


def _mesh():
    return plsc.VectorSubcoreMesh(
        core_axis_name="c", subcore_axis_name="s", num_cores=NC, num_subcores=NT
    )


def _deg_body(dst_hbm, out_hbm, idx_v, ones_v, buf_v, acc_s):
    cid = lax.axis_index("c")
    sid = lax.axis_index("s")
    wid = sid * NC + cid
    for j in range(C // 16):
        ones_v[pl.ds(j * 16, 16)] = jnp.ones((16,), jnp.float32)
    for j in range(RPT // 16):
        buf_v[pl.ds(j * 16, 16)] = jnp.zeros((16,), jnp.float32)
    pltpu.sync_copy(buf_v, acc_s.at[pl.ds(sid * RPT, RPT)])
    pltpu.sync_copy(dst_hbm.at[pl.ds(wid * K, K)], idx_v)
    plsc.subcore_barrier()

    def body(j, carry):
        pltpu.sync_copy(ones_v, acc_s.at[idx_v.at[j]], add=True)
        return carry

    lax.fori_loop(0, K, body, 0)
    plsc.subcore_barrier()
    pltpu.sync_copy(acc_s.at[pl.ds(sid * RPT, RPT)], buf_v)
    pltpu.sync_copy(buf_v, out_hbm.at[cid, pl.ds(sid * RPT, RPT)])


def _deg_call(dst2d):
    fn = functools.partial(
        pl.kernel,
        out_type=jax.ShapeDtypeStruct((NC, ACC), jnp.float32),
        mesh=_mesh(),
        scratch_types=[
            pltpu.VMEM((K, C), jnp.int32),
            pltpu.VMEM((C,), jnp.float32),
            pltpu.VMEM((RPT,), jnp.float32),
            pltpu.VMEM_SHARED((ACC,), jnp.float32),
        ],
    )(_deg_body)
    return fn(dst2d)


HALFN = N // NC
ACC_R = 5120
RPT_R = ACC_R // NT
KT = NW * K // NT
GRP = 40


def _msg_body(g_hbm, src_hbm, dst3_hbm, out_hbm, si_v, di_v, b0, b1, b2, b3,
              z_v, acc_s, s0, s1, s2, s3):
    cid = lax.axis_index("c")
    sid = lax.axis_index("s")
    for i in range(16):
        for j in range(HID // 16):
            z_v[i, pl.ds(j * 16, 16)] = jnp.zeros((16,), jnp.float32)

    def zbody(k, carry):
        pltpu.sync_copy(z_v, acc_s.at[pl.ds(sid * RPT_R + k * 16, 16)])
        return carry

    lax.fori_loop(0, RPT_R // 16, zbody, 0)
    plsc.subcore_barrier()

    bufs = (b0, b1, b2, b3)
    sems = (s0, s1, s2, s3)
    NBUF = 4
    for grp in range(KT // GRP):
        base_row = sid * KT + grp * GRP
        pltpu.sync_copy(src_hbm.at[pl.ds(base_row, GRP)], si_v)
        pltpu.sync_copy(dst3_hbm.at[cid, pl.ds(base_row, GRP)], di_v)
        for t in range(NBUF):
            pltpu.async_copy(g_hbm.at[si_v.at[t]], bufs[t], sems[t])

        @pl.loop(0, GRP, step=NBUF)
        def _(j):
            for t in range(NBUF):
                pltpu.make_async_copy(
                    g_hbm.at[pl.ds(0, C)], bufs[t], sems[t]).wait()
                pltpu.sync_copy(bufs[t], acc_s.at[di_v.at[j + t]], add=True)

                @pl.when(j + NBUF + t < GRP)
                def _():
                    pltpu.async_copy(g_hbm.at[si_v.at[j + NBUF + t]], bufs[t],
                                     sems[t])

    plsc.subcore_barrier()

    off = 0
    for sz in (C, C, RPT_R - 2 * C):
        pltpu.sync_copy(acc_s.at[pl.ds(sid * RPT_R + off, sz)],
                        b0.at[pl.ds(0, sz)])
        pltpu.sync_copy(b0.at[pl.ds(0, sz)],
                        out_hbm.at[cid, pl.ds(sid * RPT_R + off, sz)])
        off += sz


def _msg_call(g, src2d, dst3):
    fn = functools.partial(
        pl.kernel,
        out_type=jax.ShapeDtypeStruct((NC, ACC_R, HID), jnp.float32),
        mesh=_mesh(),
        scratch_types=[
            pltpu.VMEM((GRP, C), jnp.int32),
            pltpu.VMEM((GRP, C), jnp.int32),
            pltpu.VMEM((C, HID), jnp.float32),
            pltpu.VMEM((C, HID), jnp.float32),
            pltpu.VMEM((C, HID), jnp.float32),
            pltpu.VMEM((C, HID), jnp.float32),
            pltpu.VMEM((16, HID), jnp.float32),
            pltpu.VMEM_SHARED((ACC_R, HID), jnp.float32),
            pltpu.SemaphoreType.DMA,
            pltpu.SemaphoreType.DMA,
            pltpu.SemaphoreType.DMA,
            pltpu.SemaphoreType.DMA,
        ],
    )(_msg_body)
    return fn(g, src2d, dst3)


def _expand_kernel(x_ref, we_ref, be_ref, wg0_ref, degt_ref,
                   xs_ref, sum_ref, h_ref, g_ref):
    i = pl.program_id(0)
    xs = jnp.dot(x_ref[...], we_ref[...], preferred_element_type=jnp.float32)
    xs = xs + be_ref[...]
    xs_ref[...] = xs

    @pl.when(i == 0)
    def _():
        sum_ref[...] = jnp.zeros_like(sum_ref)

    sum_ref[...] += jnp.sum(xs, axis=0, keepdims=True)
    v = xs.reshape(B, NS, HID)
    x_agg = jnp.mean(v, axis=1)
    h = jnp.dot(x_agg, wg0_ref[...], preferred_element_type=jnp.float32)
    h_ref[...] = h
    deg = degt_ref[:, 0] + degt_ref[:, 1] + 1.0
    dinv = lax.rsqrt(jnp.maximum(deg, 1.0))
    g_ref[...] = h * dinv[:, None]


def _expand_call(x, W_expand, b2, Wg0, degt):
    return pl.pallas_call(
        _expand_kernel,
        grid=(NB,),
        in_specs=[
            pl.BlockSpec((B, HID), lambda i: (i, 0)),
            pl.BlockSpec((HID, NS * HID), lambda i: (0, 0)),
            pl.BlockSpec((1, NS * HID), lambda i: (0, 0)),
            pl.BlockSpec((HID, HID), lambda i: (0, 0)),
            pl.BlockSpec((B, NC), lambda i: (i, 0)),
        ],
        out_specs=[
            pl.BlockSpec((B, NS * HID), lambda i: (i, 0)),
            pl.BlockSpec((1, NS * HID), lambda i: (0, 0)),
            pl.BlockSpec((B, HID), lambda i: (i, 0)),
            pl.BlockSpec((B, HID), lambda i: (i, 0)),
        ],
        out_shape=[
            jax.ShapeDtypeStruct((N, NS * HID), jnp.float32),
            jax.ShapeDtypeStruct((1, NS * HID), jnp.float32),
            jax.ShapeDtypeStruct((N, HID), jnp.float32),
            jax.ShapeDtypeStruct((N, HID), jnp.float32),
        ],
    )(x, W_expand, b2, Wg0, degt)


def _mix_kernel(last, agg_ref, h_ref, degt_ref, xs_ref, sumxs_ref, bg_ref,
                gam_ref, bet_ref, sres_ref, spost_ref, wdr_ref, wdp_ref,
                wn_ref, bn2_ref, *rest):
    if last:
        (out_ref, r_scr, st_scr, hres_scr, hpost_scr, mu_scr) = rest
    else:
        (xsn_ref, sumn_ref, hn_ref, gn_ref,
         r_scr, st_scr, hres_scr, hpost_scr, mu_scr) = rest
    p = pl.program_id(0)
    i = pl.program_id(1)

    @pl.when(p == 0)
    def _phase0():
        deg = degt_ref[:, 0] + degt_ref[:, 1] + 1.0
        dinv = lax.rsqrt(jnp.maximum(deg, 1.0))
        agg = agg_ref[0]
        r = dinv[:, None] * agg + h_ref[...] * (dinv * dinv)[:, None] + bg_ref[...]
        r = jnp.maximum(r, 0.0)
        r_scr[pl.ds(i * B, B), :] = r

        @pl.when(i == 0)
        def _():
            st_scr[...] = jnp.zeros_like(st_scr)

        st_scr[0:1, :] += jnp.sum(r, axis=0, keepdims=True)
        st_scr[1:2, :] += jnp.sum(r * r, axis=0, keepdims=True)

        @pl.when(i == NB - 1)
        def _fin():
            mean = st_scr[0:1, :] * (1.0 / N)
            ex2 = st_scr[1:2, :] * (1.0 / N)
            var = ex2 - mean * mean
            mu_scr[0:1, :] = mean
            mu_scr[1:2, :] = lax.rsqrt(var + 1e-5)
            nxm = sumxs_ref[...].reshape(NS, HID) * (1.0 / N)
            nrm = jnp.sqrt(jnp.sum(nxm * nxm, axis=1, keepdims=True)) + 1e-6
            nx = nxm / nrm
            dyn_res = jnp.tanh(
                jnp.dot(nx, wdr_ref[...], preferred_element_type=jnp.float32))
            dyn_post = jnp.tanh(
                jnp.dot(nx, wdp_ref[...], preferred_element_type=jnp.float32))
            M = jnp.exp((sres_ref[...] + dyn_res) * (1.0 / TAU))
            for _ in range(SINK_ITERS):
                M = M / (jnp.sum(M, axis=1, keepdims=True) + 1e-8)
                M = M / (jnp.sum(M, axis=0, keepdims=True) + 1e-8)
            hres_scr[...] = M
            z = spost_ref[...] + dyn_post
            hpost_scr[...] = 1.0 / (1.0 + jnp.exp(-z))

    @pl.when(p == 1)
    def _phase1():
        r = r_scr[pl.ds(i * B, B), :]
        m = gam_ref[...] * (r - mu_scr[0:1, :]) * mu_scr[1:2, :] + bet_ref[...]
        v = xs_ref[...].reshape(B, NS, HID)
        H = hres_scr[...]
        hp = hpost_scr[...]
        outs = []
        for a in range(NS):
            acc = hp[a, 0] * m
            for b in range(NS):
                acc = acc + H[a, b] * v[:, b, :]
            outs.append(acc)
        x_agg = (outs[0] + outs[1] + outs[2] + outs[3]) * (1.0 / NS)
        if last:
            out_ref[...] = jnp.dot(
                x_agg, wn_ref[...], preferred_element_type=jnp.float32
            ) + bn2_ref[...]
        else:
            xs_new = jnp.stack(outs, axis=1)
            xsn_ref[...] = xs_new.reshape(B, NS * HID)

            @pl.when(i == 0)
            def _():
                sumn_ref[...] = jnp.zeros_like(sumn_ref)

            sumn_ref[...] += jnp.sum(xs_new, axis=0).reshape(1, NS * HID)
            hn = jnp.dot(x_agg, wn_ref[...], preferred_element_type=jnp.float32)
            hn_ref[...] = hn
            deg = degt_ref[:, 0] + degt_ref[:, 1] + 1.0
            dinv = lax.rsqrt(jnp.maximum(deg, 1.0))
            gn_ref[...] = hn * dinv[:, None]


def _mix_call(last, agg, h, degt, xs, sumxs, bg, gam, bet, sres, spost, wdr,
              wdp, wn, bn2):
    wn_cols = wn.shape[1]
    in_specs = [
        pl.BlockSpec((1, B, HID),
                     lambda p, i: ((1 - p) * (i // (NB // NC)),
                                   (1 - p) * (i % (NB // NC)), 0)),
        pl.BlockSpec((B, HID), lambda p, i: (i * (1 - p), 0)),
        pl.BlockSpec((B, NC), lambda p, i: (i, 0)),
        pl.BlockSpec((B, NS * HID), lambda p, i: (i * p, 0)),
        pl.BlockSpec((1, NS * HID), lambda p, i: (0, 0)),
        pl.BlockSpec((1, HID), lambda p, i: (0, 0)),
        pl.BlockSpec((1, HID), lambda p, i: (0, 0)),
        pl.BlockSpec((1, HID), lambda p, i: (0, 0)),
        pl.BlockSpec((NS, NS), lambda p, i: (0, 0)),
        pl.BlockSpec((NS, 1), lambda p, i: (0, 0)),
        pl.BlockSpec((HID, NS), lambda p, i: (0, 0)),
        pl.BlockSpec((HID, 1), lambda p, i: (0, 0)),
        pl.BlockSpec((HID, wn_cols), lambda p, i: (0, 0)),
        pl.BlockSpec((1, wn_cols), lambda p, i: (0, 0)),
    ]
    if last:
        out_specs = [pl.BlockSpec((B, wn_cols), lambda p, i: (i * p, 0))]
        out_shape = [jax.ShapeDtypeStruct((N, wn_cols), jnp.float32)]
    else:
        out_specs = [
            pl.BlockSpec((B, NS * HID), lambda p, i: (i * p, 0)),
            pl.BlockSpec((1, NS * HID), lambda p, i: (0, 0)),
            pl.BlockSpec((B, HID), lambda p, i: (i * p, 0)),
            pl.BlockSpec((B, HID), lambda p, i: (i * p, 0)),
        ]
        out_shape = [
            jax.ShapeDtypeStruct((N, NS * HID), jnp.float32),
            jax.ShapeDtypeStruct((1, NS * HID), jnp.float32),
            jax.ShapeDtypeStruct((N, HID), jnp.float32),
            jax.ShapeDtypeStruct((N, HID), jnp.float32),
        ]
    return pl.pallas_call(
        functools.partial(_mix_kernel, last),
        grid=(2, NB),
        in_specs=in_specs,
        out_specs=out_specs,
        out_shape=out_shape,
        scratch_shapes=[
            pltpu.VMEM((N, HID), jnp.float32),
            pltpu.VMEM((2, HID), jnp.float32),
            pltpu.VMEM((NS, NS), jnp.float32),
            pltpu.VMEM((NS, 1), jnp.float32),
            pltpu.VMEM((2, HID), jnp.float32),
        ],
    )(agg, h, degt, xs, sumxs, bg, gam, bet, sres, spost, wdr, wdp, wn, bn2)


def kernel(x, edge_index, W_expand, b_expand, W_gcn, b_gcn, bn_gamma, bn_beta,
           static_res, static_post, W_dyn_res, w_dyn_post, W_out, b_out):
    src = edge_index[0]
    dst = edge_index[1]
    pad = EP - E
    src2d = jnp.concatenate([src, jnp.zeros((pad,), jnp.int32)]).reshape(NW * K, C)
    dstp = jnp.concatenate([dst, jnp.full((pad,), N, jnp.int32)])
    dst2d = dstp.reshape(NW * K, C)
    spread = HALFN + (jnp.arange(EP, dtype=jnp.int32) % (ACC_R - HALFN))
    dst3 = []
    for c in range(NC):
        t = dstp - c * HALFN
        ok = (t >= 0) & (t < HALFN)
        dst3.append(jnp.where(ok, t, spread))
    dst3 = jnp.stack(dst3).reshape(NC, NW * K, C)

    degp = _deg_call(dst2d)
    degt = jnp.transpose(degp)
    xs, sumxs, h, g = _expand_call(
        x, W_expand, b_expand.reshape(1, NS * HID), W_gcn[0], degt)

    for l in range(3):
        agg = _msg_call(g, src2d, dst3)
        last = l == 2
        wn = W_out if last else W_gcn[l + 1]
        bn2 = b_out.reshape(1, -1) if last else jnp.zeros((1, HID), jnp.float32)
        res = _mix_call(
            last, agg, h, degt, xs, sumxs,
            b_gcn[l].reshape(1, HID),
            bn_gamma[l].reshape(1, HID),
            bn_beta[l].reshape(1, HID),
            static_res[l],
            static_post[l].reshape(NS, 1),
            W_dyn_res[l],
            w_dyn_post[l].reshape(HID, 1),
            wn, bn2)
        if last:
            return res[0]
        xs, sumxs, h, g = res


# speedup vs baseline: 6.8298x; 1.5125x over previous
import functools

import jax
import jax.numpy as jnp
from jax import lax
from jax.experimental import pallas as pl
from jax.experimental.pallas import tpu as pltpu
from jax.experimental.pallas import tpu_sc as plsc

N = 10000
E = 320000
HID = 128
NS = 4
TAU = 0.1
SINK_ITERS = 10

NC = 2
NT = 16
NW = NC * NT

C = 128
K = 80
EP = NW * K * C
ACC = 10240
RPT = ACC // NT

B = 1000
NB = N // B


def _mesh():
    return plsc.VectorSubcoreMesh(
        core_axis_name="c", subcore_axis_name="s", num_cores=NC, num_subcores=NT
    )


def _deg_body(dst_hbm, out_hbm, idx_v, ones_v, buf_v, acc_s):
    cid = lax.axis_index("c")
    sid = lax.axis_index("s")
    wid = sid * NC + cid
    for j in range(C // 16):
        ones_v[pl.ds(j * 16, 16)] = jnp.ones((16,), jnp.float32)
    for j in range(RPT // 16):
        buf_v[pl.ds(j * 16, 16)] = jnp.zeros((16,), jnp.float32)
    pltpu.sync_copy(buf_v, acc_s.at[pl.ds(sid * RPT, RPT)])
    pltpu.sync_copy(dst_hbm.at[pl.ds(wid * K, K)], idx_v)
    plsc.subcore_barrier()

    def body(j, carry):
        pltpu.sync_copy(ones_v, acc_s.at[idx_v.at[j]], add=True)
        return carry

    lax.fori_loop(0, K, body, 0)
    plsc.subcore_barrier()
    pltpu.sync_copy(acc_s.at[pl.ds(sid * RPT, RPT)], buf_v)
    pltpu.sync_copy(buf_v, out_hbm.at[cid, pl.ds(sid * RPT, RPT)])


def _deg_call(dst2d):
    fn = functools.partial(
        pl.kernel,
        out_type=jax.ShapeDtypeStruct((NC, ACC), jnp.float32),
        mesh=_mesh(),
        scratch_types=[
            pltpu.VMEM((K, C), jnp.int32),
            pltpu.VMEM((C,), jnp.float32),
            pltpu.VMEM((RPT,), jnp.float32),
            pltpu.VMEM_SHARED((ACC,), jnp.float32),
        ],
    )(_deg_body)
    return fn(dst2d)


GRP = 16
NBUF = 2


def _msg_body(g_hbm, src_hbm, dst_hbm, out_hbm, si_v, di_v, b0, b1,
              z_v, acc_s, s0, s1):
    cid = lax.axis_index("c")
    sid = lax.axis_index("s")
    wid = sid * NC + cid
    for i in range(16):
        for j in range(HID // 16):
            z_v[i, pl.ds(j * 16, 16)] = jnp.zeros((16,), jnp.float32)

    def zbody(k, carry):
        pltpu.sync_copy(z_v, acc_s.at[pl.ds(sid * RPT + k * 16, 16)])
        return carry

    lax.fori_loop(0, RPT // 16, zbody, 0)
    plsc.subcore_barrier()

    bufs = (b0, b1)
    sems = (s0, s1)
    for grp in range(K // GRP):
        base_row = wid * K + grp * GRP
        pltpu.sync_copy(src_hbm.at[pl.ds(base_row, GRP)], si_v)
        pltpu.sync_copy(dst_hbm.at[pl.ds(base_row, GRP)], di_v)
        for t in range(NBUF):
            pltpu.async_copy(g_hbm.at[si_v.at[t]], bufs[t], sems[t])

        @pl.loop(0, GRP, step=NBUF)
        def _(j):
            for t in range(NBUF):
                pltpu.make_async_copy(
                    g_hbm.at[pl.ds(0, C)], bufs[t], sems[t]).wait()
                pltpu.sync_copy(bufs[t], acc_s.at[di_v.at[j + t]], add=True)

                @pl.when(j + NBUF + t < GRP)
                def _():
                    pltpu.async_copy(g_hbm.at[si_v.at[j + NBUF + t]], bufs[t],
                                     sems[t])

    plsc.subcore_barrier()

    def wbody(k, carry):
        pltpu.sync_copy(acc_s.at[pl.ds(sid * RPT + k * C, C)], b0)
        pltpu.sync_copy(b0, out_hbm.at[cid, pl.ds(sid * RPT + k * C, C)])
        return carry

    lax.fori_loop(0, RPT // C, wbody, 0)


def _msg_call(g, src2d, dst2d):
    fn = functools.partial(
        pl.kernel,
        out_type=jax.ShapeDtypeStruct((NC, ACC, HID), jnp.float32),
        mesh=_mesh(),
        scratch_types=[
            pltpu.VMEM((GRP, C), jnp.int32),
            pltpu.VMEM((GRP, C), jnp.int32),
            pltpu.VMEM((C, HID), jnp.float32),
            pltpu.VMEM((C, HID), jnp.float32),
            pltpu.VMEM((16, HID), jnp.float32),
            pltpu.VMEM_SHARED((ACC, HID), jnp.float32),
            pltpu.SemaphoreType.DMA,
            pltpu.SemaphoreType.DMA,
        ],
    )(_msg_body)
    return fn(g, src2d, dst2d)


def _expand_kernel(x_ref, we_ref, be_ref, wg0_ref, degt_ref,
                   xs_ref, sum_ref, h_ref, g_ref):
    i = pl.program_id(0)
    xs = jnp.dot(x_ref[...], we_ref[...], preferred_element_type=jnp.float32)
    xs = xs + be_ref[...]
    xs_ref[...] = xs

    @pl.when(i == 0)
    def _():
        sum_ref[...] = jnp.zeros_like(sum_ref)

    sum_ref[...] += jnp.sum(xs, axis=0, keepdims=True)
    v = xs.reshape(B, NS, HID)
    x_agg = jnp.mean(v, axis=1)
    h = jnp.dot(x_agg, wg0_ref[...], preferred_element_type=jnp.float32)
    h_ref[...] = h
    deg = degt_ref[:, 0] + degt_ref[:, 1] + 1.0
    dinv = lax.rsqrt(jnp.maximum(deg, 1.0))
    g_ref[...] = h * dinv[:, None]


def _expand_call(x, W_expand, b2, Wg0, degt):
    return pl.pallas_call(
        _expand_kernel,
        grid=(NB,),
        in_specs=[
            pl.BlockSpec((B, HID), lambda i: (i, 0)),
            pl.BlockSpec((HID, NS * HID), lambda i: (0, 0)),
            pl.BlockSpec((1, NS * HID), lambda i: (0, 0)),
            pl.BlockSpec((HID, HID), lambda i: (0, 0)),
            pl.BlockSpec((B, NC), lambda i: (i, 0)),
        ],
        out_specs=[
            pl.BlockSpec((B, NS * HID), lambda i: (i, 0)),
            pl.BlockSpec((1, NS * HID), lambda i: (0, 0)),
            pl.BlockSpec((B, HID), lambda i: (i, 0)),
            pl.BlockSpec((B, HID), lambda i: (i, 0)),
        ],
        out_shape=[
            jax.ShapeDtypeStruct((N, NS * HID), jnp.float32),
            jax.ShapeDtypeStruct((1, NS * HID), jnp.float32),
            jax.ShapeDtypeStruct((N, HID), jnp.float32),
            jax.ShapeDtypeStruct((N, HID), jnp.float32),
        ],
    )(x, W_expand, b2, Wg0, degt)


def _mix_kernel(last, agg_ref, h_ref, degt_ref, xs_ref, sumxs_ref, bg_ref,
                gam_ref, bet_ref, sres_ref, spost_ref, wdr_ref, wdp_ref,
                wn_ref, bn2_ref, *rest):
    if last:
        (out_ref, r_scr, st_scr, hres_scr, hpost_scr, mu_scr) = rest
    else:
        (xsn_ref, sumn_ref, hn_ref, gn_ref,
         r_scr, st_scr, hres_scr, hpost_scr, mu_scr) = rest
    p = pl.program_id(0)
    i = pl.program_id(1)

    @pl.when(p == 0)
    def _phase0():
        deg = degt_ref[:, 0] + degt_ref[:, 1] + 1.0
        dinv = lax.rsqrt(jnp.maximum(deg, 1.0))
        agg = agg_ref[0] + agg_ref[1]
        r = dinv[:, None] * agg + h_ref[...] * (dinv * dinv)[:, None] + bg_ref[...]
        r = jnp.maximum(r, 0.0)
        r_scr[pl.ds(i * B, B), :] = r

        @pl.when(i == 0)
        def _():
            st_scr[...] = jnp.zeros_like(st_scr)

        st_scr[0:1, :] += jnp.sum(r, axis=0, keepdims=True)
        st_scr[1:2, :] += jnp.sum(r * r, axis=0, keepdims=True)

        @pl.when(i == NB - 1)
        def _fin():
            mean = st_scr[0:1, :] * (1.0 / N)
            ex2 = st_scr[1:2, :] * (1.0 / N)
            var = ex2 - mean * mean
            mu_scr[0:1, :] = mean
            mu_scr[1:2, :] = lax.rsqrt(var + 1e-5)
            nxm = sumxs_ref[...].reshape(NS, HID) * (1.0 / N)
            nrm = jnp.sqrt(jnp.sum(nxm * nxm, axis=1, keepdims=True)) + 1e-6
            nx = nxm / nrm
            dyn_res = jnp.tanh(
                jnp.dot(nx, wdr_ref[...], preferred_element_type=jnp.float32))
            dyn_post = jnp.tanh(
                jnp.dot(nx, wdp_ref[...], preferred_element_type=jnp.float32))
            M = jnp.exp((sres_ref[...] + dyn_res) * (1.0 / TAU))
            for _ in range(SINK_ITERS):
                M = M / (jnp.sum(M, axis=1, keepdims=True) + 1e-8)
                M = M / (jnp.sum(M, axis=0, keepdims=True) + 1e-8)
            hres_scr[...] = M
            z = spost_ref[...] + dyn_post
            hpost_scr[...] = 1.0 / (1.0 + jnp.exp(-z))

    @pl.when(p == 1)
    def _phase1():
        r = r_scr[pl.ds(i * B, B), :]
        m = gam_ref[...] * (r - mu_scr[0:1, :]) * mu_scr[1:2, :] + bet_ref[...]
        v = xs_ref[...].reshape(B, NS, HID)
        H = hres_scr[...]
        hp = hpost_scr[...]
        outs = []
        for a in range(NS):
            acc = hp[a, 0] * m
            for b in range(NS):
                acc = acc + H[a, b] * v[:, b, :]
            outs.append(acc)
        x_agg = (outs[0] + outs[1] + outs[2] + outs[3]) * (1.0 / NS)
        if last:
            out_ref[...] = jnp.dot(
                x_agg, wn_ref[...], preferred_element_type=jnp.float32
            ) + bn2_ref[...]
        else:
            xs_new = jnp.stack(outs, axis=1)
            xsn_ref[...] = xs_new.reshape(B, NS * HID)

            @pl.when(i == 0)
            def _():
                sumn_ref[...] = jnp.zeros_like(sumn_ref)

            sumn_ref[...] += jnp.sum(xs_new, axis=0).reshape(1, NS * HID)
            hn = jnp.dot(x_agg, wn_ref[...], preferred_element_type=jnp.float32)
            hn_ref[...] = hn
            deg = degt_ref[:, 0] + degt_ref[:, 1] + 1.0
            dinv = lax.rsqrt(jnp.maximum(deg, 1.0))
            gn_ref[...] = hn * dinv[:, None]


def _mix_call(last, agg, h, degt, xs, sumxs, bg, gam, bet, sres, spost, wdr,
              wdp, wn, bn2):
    wn_cols = wn.shape[1]
    in_specs = [
        pl.BlockSpec((NC, B, HID), lambda p, i: (0, i * (1 - p), 0)),
        pl.BlockSpec((B, HID), lambda p, i: (i * (1 - p), 0)),
        pl.BlockSpec((B, NC), lambda p, i: (i, 0)),
        pl.BlockSpec((B, NS * HID), lambda p, i: (i * p, 0)),
        pl.BlockSpec((1, NS * HID), lambda p, i: (0, 0)),
        pl.BlockSpec((1, HID), lambda p, i: (0, 0)),
        pl.BlockSpec((1, HID), lambda p, i: (0, 0)),
        pl.BlockSpec((1, HID), lambda p, i: (0, 0)),
        pl.BlockSpec((NS, NS), lambda p, i: (0, 0)),
        pl.BlockSpec((NS, 1), lambda p, i: (0, 0)),
        pl.BlockSpec((HID, NS), lambda p, i: (0, 0)),
        pl.BlockSpec((HID, 1), lambda p, i: (0, 0)),
        pl.BlockSpec((HID, wn_cols), lambda p, i: (0, 0)),
        pl.BlockSpec((1, wn_cols), lambda p, i: (0, 0)),
    ]
    if last:
        out_specs = [pl.BlockSpec((B, wn_cols), lambda p, i: (i * p, 0))]
        out_shape = [jax.ShapeDtypeStruct((N, wn_cols), jnp.float32)]
    else:
        out_specs = [
            pl.BlockSpec((B, NS * HID), lambda p, i: (i * p, 0)),
            pl.BlockSpec((1, NS * HID), lambda p, i: (0, 0)),
            pl.BlockSpec((B, HID), lambda p, i: (i * p, 0)),
            pl.BlockSpec((B, HID), lambda p, i: (i * p, 0)),
        ]
        out_shape = [
            jax.ShapeDtypeStruct((N, NS * HID), jnp.float32),
            jax.ShapeDtypeStruct((1, NS * HID), jnp.float32),
            jax.ShapeDtypeStruct((N, HID), jnp.float32),
            jax.ShapeDtypeStruct((N, HID), jnp.float32),
        ]
    return pl.pallas_call(
        functools.partial(_mix_kernel, last),
        grid=(2, NB),
        in_specs=in_specs,
        out_specs=out_specs,
        out_shape=out_shape,
        scratch_shapes=[
            pltpu.VMEM((N, HID), jnp.float32),
            pltpu.VMEM((2, HID), jnp.float32),
            pltpu.VMEM((NS, NS), jnp.float32),
            pltpu.VMEM((NS, 1), jnp.float32),
            pltpu.VMEM((2, HID), jnp.float32),
        ],
    )(agg, h, degt, xs, sumxs, bg, gam, bet, sres, spost, wdr, wdp, wn, bn2)


def kernel(x, edge_index, W_expand, b_expand, W_gcn, b_gcn, bn_gamma, bn_beta,
           static_res, static_post, W_dyn_res, w_dyn_post, W_out, b_out):
    src = edge_index[0]
    dst = edge_index[1]
    pad = EP - E
    src2d = jnp.concatenate([src, jnp.zeros((pad,), jnp.int32)]).reshape(NW * K, C)
    dst2d = jnp.concatenate([dst, jnp.full((pad,), N, jnp.int32)]).reshape(NW * K, C)

    degp = _deg_call(dst2d)
    degt = jnp.transpose(degp)
    xs, sumxs, h, g = _expand_call(
        x, W_expand, b_expand.reshape(1, NS * HID), W_gcn[0], degt)

    for l in range(3):
        agg = _msg_call(g, src2d, dst2d)
        last = l == 2
        wn = W_out if last else W_gcn[l + 1]
        bn2 = b_out.reshape(1, -1) if last else jnp.zeros((1, HID), jnp.float32)
        res = _mix_call(
            last, agg, h, degt, xs, sumxs,
            b_gcn[l].reshape(1, HID),
            bn_gamma[l].reshape(1, HID),
            bn_beta[l].reshape(1, HID),
            static_res[l],
            static_post[l].reshape(NS, 1),
            W_dyn_res[l],
            w_dyn_post[l].reshape(HID, 1),
            wn, bn2)
        if last:
            return res[0]
        xs, sumxs, h, g = res


# speedup vs baseline: 6.8388x; 1.0013x over previous
import functools

import jax
import jax.numpy as jnp
from jax import lax
from jax.experimental import pallas as pl
from jax.experimental.pallas import tpu as pltpu
from jax.experimental.pallas import tpu_sc as plsc

N = 10000
E = 320000
HID = 128
NS = 4
TAU = 0.1
SINK_ITERS = 10

NC = 2
NT = 16
NW = NC * NT

C = 128
K = 80
EP = NW * K * C
ACC = 10240
RPT = ACC // NT

B = 1000
NB = N // B


def _mesh():
    return plsc.VectorSubcoreMesh(
        core_axis_name="c", subcore_axis_name="s", num_cores=NC, num_subcores=NT
    )


def _deg_body(dst_hbm, out_hbm, idx_v, ones_v, buf_v, acc_s):
    cid = lax.axis_index("c")
    sid = lax.axis_index("s")
    wid = sid * NC + cid
    for j in range(C // 16):
        ones_v[pl.ds(j * 16, 16)] = jnp.ones((16,), jnp.float32)
    for j in range(RPT // 16):
        buf_v[pl.ds(j * 16, 16)] = jnp.zeros((16,), jnp.float32)
    pltpu.sync_copy(buf_v, acc_s.at[pl.ds(sid * RPT, RPT)])
    pltpu.sync_copy(dst_hbm.at[pl.ds(wid * K, K)], idx_v)
    plsc.subcore_barrier()

    def body(j, carry):
        pltpu.sync_copy(ones_v, acc_s.at[idx_v.at[j]], add=True)
        return carry

    lax.fori_loop(0, K, body, 0)
    plsc.subcore_barrier()
    pltpu.sync_copy(acc_s.at[pl.ds(sid * RPT, RPT)], buf_v)
    pltpu.sync_copy(buf_v, out_hbm.at[cid, pl.ds(sid * RPT, RPT)])


def _deg_call(dst2d):
    fn = functools.partial(
        pl.kernel,
        out_type=jax.ShapeDtypeStruct((NC, ACC), jnp.float32),
        mesh=_mesh(),
        scratch_types=[
            pltpu.VMEM((K, C), jnp.int32),
            pltpu.VMEM((C,), jnp.float32),
            pltpu.VMEM((RPT,), jnp.float32),
            pltpu.VMEM_SHARED((ACC,), jnp.float32),
        ],
    )(_deg_body)
    return fn(dst2d)


GRP = 16
NBUF = 2


def _msg_body(g_hbm, src_hbm, dst_hbm, out_hbm, si_v, di_v, b0, b1,
              z_v, acc_s, s0, s1):
    cid = lax.axis_index("c")
    sid = lax.axis_index("s")
    wid = sid * NC + cid
    for i in range(16):
        for j in range(HID // 16):
            z_v[i, pl.ds(j * 16, 16)] = jnp.zeros((16,), jnp.float32)

    def zbody(k, carry):
        pltpu.sync_copy(z_v, acc_s.at[pl.ds(sid * RPT + k * 16, 16)])
        return carry

    lax.fori_loop(0, RPT // 16, zbody, 0)
    plsc.subcore_barrier()

    bufs = (b0, b1)
    sems = (s0, s1)
    for grp in range(K // GRP):
        base_row = wid * K + grp * GRP
        pltpu.sync_copy(src_hbm.at[pl.ds(base_row, GRP)], si_v)
        pltpu.sync_copy(dst_hbm.at[pl.ds(base_row, GRP)], di_v)
        for t in range(NBUF):
            pltpu.async_copy(g_hbm.at[si_v.at[t]], bufs[t], sems[t])

        @pl.loop(0, GRP, step=NBUF)
        def _(j):
            for t in range(NBUF):
                pltpu.make_async_copy(
                    g_hbm.at[pl.ds(0, C)], bufs[t], sems[t]).wait()
                pltpu.sync_copy(bufs[t], acc_s.at[di_v.at[j + t]], add=True)

                @pl.when(j + NBUF + t < GRP)
                def _():
                    pltpu.async_copy(g_hbm.at[si_v.at[j + NBUF + t]], bufs[t],
                                     sems[t])

    plsc.subcore_barrier()

    def wbody(k, carry):
        pltpu.sync_copy(acc_s.at[pl.ds(sid * RPT + k * C, C)], b0)
        pltpu.sync_copy(b0, out_hbm.at[cid, pl.ds(sid * RPT + k * C, C)])
        return carry

    lax.fori_loop(0, RPT // C, wbody, 0)


def _msg_call(g, src2d, dst2d):
    fn = functools.partial(
        pl.kernel,
        out_type=jax.ShapeDtypeStruct((NC, ACC, HID), jnp.float32),
        mesh=_mesh(),
        scratch_types=[
            pltpu.VMEM((GRP, C), jnp.int32),
            pltpu.VMEM((GRP, C), jnp.int32),
            pltpu.VMEM((C, HID), jnp.float32),
            pltpu.VMEM((C, HID), jnp.float32),
            pltpu.VMEM((16, HID), jnp.float32),
            pltpu.VMEM_SHARED((ACC, HID), jnp.float32),
            pltpu.SemaphoreType.DMA,
            pltpu.SemaphoreType.DMA,
        ],
    )(_msg_body)
    return fn(g, src2d, dst2d)


def _expand_kernel(x_ref, we_ref, be_ref, wg0_ref, degt_ref,
                   xs_ref, sum_ref, h_ref, g_ref):
    i = pl.program_id(0)
    xs = jnp.dot(x_ref[...], we_ref[...], preferred_element_type=jnp.float32)
    xs = xs + be_ref[...]
    xs_ref[...] = xs

    @pl.when(i == 0)
    def _():
        sum_ref[...] = jnp.zeros_like(sum_ref)

    sum_ref[...] += jnp.sum(xs, axis=0, keepdims=True)
    v = xs.reshape(B, NS, HID)
    x_agg = jnp.mean(v, axis=1)
    h = jnp.dot(x_agg, wg0_ref[...], preferred_element_type=jnp.float32)
    h_ref[...] = h
    deg = degt_ref[:, 0] + degt_ref[:, 1] + 1.0
    dinv = lax.rsqrt(jnp.maximum(deg, 1.0))
    g_ref[...] = h * dinv[:, None]


def _expand_call(x, W_expand, b2, Wg0, degt):
    return pl.pallas_call(
        _expand_kernel,
        grid=(NB,),
        in_specs=[
            pl.BlockSpec((B, HID), lambda i: (i, 0)),
            pl.BlockSpec((HID, NS * HID), lambda i: (0, 0)),
            pl.BlockSpec((1, NS * HID), lambda i: (0, 0)),
            pl.BlockSpec((HID, HID), lambda i: (0, 0)),
            pl.BlockSpec((B, NC), lambda i: (i, 0)),
        ],
        out_specs=[
            pl.BlockSpec((B, NS * HID), lambda i: (i, 0)),
            pl.BlockSpec((1, NS * HID), lambda i: (0, 0)),
            pl.BlockSpec((B, HID), lambda i: (i, 0)),
            pl.BlockSpec((B, HID), lambda i: (i, 0)),
        ],
        out_shape=[
            jax.ShapeDtypeStruct((N, NS * HID), jnp.float32),
            jax.ShapeDtypeStruct((1, NS * HID), jnp.float32),
            jax.ShapeDtypeStruct((N, HID), jnp.float32),
            jax.ShapeDtypeStruct((N, HID), jnp.float32),
        ],
    )(x, W_expand, b2, Wg0, degt)


def _mix_kernel(last, agg_ref, h_ref, degt_ref, xs_ref, sumxs_ref, bg_ref,
                gam_ref, bet_ref, sres_ref, spost_ref, wdr_ref, wdp_ref,
                wn_ref, bn2_ref, *rest):
    if last:
        (out_ref, r_scr, st_scr, hres_scr, hpost_scr, mu_scr) = rest
    else:
        (xsn_ref, sumn_ref, hn_ref, gn_ref,
         r_scr, st_scr, hres_scr, hpost_scr, mu_scr) = rest
    p = pl.program_id(0)
    i = pl.program_id(1)

    @pl.when(p == 0)
    def _phase0():
        deg = degt_ref[:, 0] + degt_ref[:, 1] + 1.0
        dinv = lax.rsqrt(jnp.maximum(deg, 1.0))
        agg = agg_ref[0] + agg_ref[1]
        r = dinv[:, None] * agg + h_ref[...] * (dinv * dinv)[:, None] + bg_ref[...]
        r = jnp.maximum(r, 0.0)
        r_scr[pl.ds(i * B, B), :] = r

        @pl.when(i == 0)
        def _():
            st_scr[...] = jnp.zeros_like(st_scr)

        st_scr[0:1, :] += jnp.sum(r, axis=0, keepdims=True)
        st_scr[1:2, :] += jnp.sum(r * r, axis=0, keepdims=True)

        @pl.when(i == NB - 1)
        def _fin():
            mean = st_scr[0:1, :] * (1.0 / N)
            ex2 = st_scr[1:2, :] * (1.0 / N)
            var = ex2 - mean * mean
            mu_scr[0:1, :] = mean
            mu_scr[1:2, :] = lax.rsqrt(var + 1e-5)
            nxm = sumxs_ref[...].reshape(NS, HID) * (1.0 / N)
            nrm = jnp.sqrt(jnp.sum(nxm * nxm, axis=1, keepdims=True)) + 1e-6
            nx = nxm / nrm
            dyn_res = jnp.tanh(
                jnp.dot(nx, wdr_ref[...], preferred_element_type=jnp.float32))
            dyn_post = jnp.tanh(
                jnp.dot(nx, wdp_ref[...], preferred_element_type=jnp.float32))
            M = jnp.exp((sres_ref[...] + dyn_res) * (1.0 / TAU))
            for _ in range(SINK_ITERS):
                M = M / (jnp.sum(M, axis=1, keepdims=True) + 1e-8)
                M = M / (jnp.sum(M, axis=0, keepdims=True) + 1e-8)
            hres_scr[...] = M
            z = spost_ref[...] + dyn_post
            hpost_scr[...] = 1.0 / (1.0 + jnp.exp(-z))

    @pl.when(p == 1)
    def _phase1():
        r = r_scr[pl.ds(i * B, B), :]
        m = gam_ref[...] * (r - mu_scr[0:1, :]) * mu_scr[1:2, :] + bet_ref[...]
        v = xs_ref[...].reshape(B, NS, HID)
        H = hres_scr[...]
        hp = hpost_scr[...]
        outs = []
        for a in range(NS):
            acc = hp[a, 0] * m
            for b in range(NS):
                acc = acc + H[a, b] * v[:, b, :]
            outs.append(acc)
        x_agg = (outs[0] + outs[1] + outs[2] + outs[3]) * (1.0 / NS)
        if last:
            out_ref[...] = jnp.dot(
                x_agg, wn_ref[...], preferred_element_type=jnp.float32
            ) + bn2_ref[...]
        else:
            xs_new = jnp.stack(outs, axis=1)
            xsn_ref[...] = xs_new.reshape(B, NS * HID)

            @pl.when(i == 0)
            def _():
                sumn_ref[...] = jnp.zeros_like(sumn_ref)

            sumn_ref[...] += jnp.sum(xs_new, axis=0).reshape(1, NS * HID)
            hn = jnp.dot(x_agg, wn_ref[...], preferred_element_type=jnp.float32)
            hn_ref[...] = hn
            deg = degt_ref[:, 0] + degt_ref[:, 1] + 1.0
            dinv = lax.rsqrt(jnp.maximum(deg, 1.0))
            gn_ref[...] = hn * dinv[:, None]


def _mix_call(last, agg, h, degt, xs, sumxs, bg, gam, bet, sres, spost, wdr,
              wdp, wn, bn2):
    wn_cols = wn.shape[1]
    in_specs = [
        pl.BlockSpec((NC, B, HID), lambda p, i: (0, i * (1 - p), 0)),
        pl.BlockSpec((B, HID), lambda p, i: (i * (1 - p), 0)),
        pl.BlockSpec((B, NC), lambda p, i: (i, 0)),
        pl.BlockSpec((B, NS * HID), lambda p, i: (i * p, 0)),
        pl.BlockSpec((1, NS * HID), lambda p, i: (0, 0)),
        pl.BlockSpec((1, HID), lambda p, i: (0, 0)),
        pl.BlockSpec((1, HID), lambda p, i: (0, 0)),
        pl.BlockSpec((1, HID), lambda p, i: (0, 0)),
        pl.BlockSpec((NS, NS), lambda p, i: (0, 0)),
        pl.BlockSpec((NS, 1), lambda p, i: (0, 0)),
        pl.BlockSpec((HID, NS), lambda p, i: (0, 0)),
        pl.BlockSpec((HID, 1), lambda p, i: (0, 0)),
        pl.BlockSpec((HID, wn_cols), lambda p, i: (0, 0)),
        pl.BlockSpec((1, wn_cols), lambda p, i: (0, 0)),
    ]
    if last:
        out_specs = [pl.BlockSpec((B, wn_cols), lambda p, i: (i * p, 0))]
        out_shape = [jax.ShapeDtypeStruct((N, wn_cols), jnp.float32)]
    else:
        out_specs = [
            pl.BlockSpec((B, NS * HID), lambda p, i: (i * p, 0)),
            pl.BlockSpec((1, NS * HID), lambda p, i: (0, 0)),
            pl.BlockSpec((B, HID), lambda p, i: (i * p, 0)),
            pl.BlockSpec((B, HID), lambda p, i: (i * p, 0)),
        ]
        out_shape = [
            jax.ShapeDtypeStruct((N, NS * HID), jnp.float32),
            jax.ShapeDtypeStruct((1, NS * HID), jnp.float32),
            jax.ShapeDtypeStruct((N, HID), jnp.float32),
            jax.ShapeDtypeStruct((N, HID), jnp.float32),
        ]
    return pl.pallas_call(
        functools.partial(_mix_kernel, last),
        grid=(2, NB),
        in_specs=in_specs,
        out_specs=out_specs,
        out_shape=out_shape,
        scratch_shapes=[
            pltpu.VMEM((N, HID), jnp.float32),
            pltpu.VMEM((2, HID), jnp.float32),
            pltpu.VMEM((NS, NS), jnp.float32),
            pltpu.VMEM((NS, 1), jnp.float32),
            pltpu.VMEM((2, HID), jnp.float32),
        ],
    )(agg, h, degt, xs, sumxs, bg, gam, bet, sres, spost, wdr, wdp, wn, bn2)


def kernel(x, edge_index, W_expand, b_expand, W_gcn, b_gcn, bn_gamma, bn_beta,
           static_res, static_post, W_dyn_res, w_dyn_post, W_out, b_out):
    src = edge_index[0]
    dst = edge_index[1]
    pad = EP - E
    src2d = jnp.concatenate([src, jnp.zeros((pad,), jnp.int32)]).reshape(NW * K, C)
    pad_dst = N + (jnp.arange(pad, dtype=jnp.int32) % (ACC - N))
    dst2d = jnp.concatenate([dst, pad_dst]).reshape(NW * K, C)

    degp = _deg_call(dst2d)
    degt = jnp.transpose(degp)
    xs, sumxs, h, g = _expand_call(
        x, W_expand, b_expand.reshape(1, NS * HID), W_gcn[0], degt)

    for l in range(3):
        agg = _msg_call(g, src2d, dst2d)
        last = l == 2
        wn = W_out if last else W_gcn[l + 1]
        bn2 = b_out.reshape(1, -1) if last else jnp.zeros((1, HID), jnp.float32)
        res = _mix_call(
            last, agg, h, degt, xs, sumxs,
            b_gcn[l].reshape(1, HID),
            bn_gamma[l].reshape(1, HID),
            bn_beta[l].reshape(1, HID),
            static_res[l],
            static_post[l].reshape(NS, 1),
            W_dyn_res[l],
            w_dyn_post[l].reshape(HID, 1),
            wn, bn2)
        if last:
            return res[0]
        xs, sumxs, h, g = res
